# baseline (speedup 1.0000x reference)
import functools

import jax
import jax.numpy as jnp
from jax import lax
from jax.experimental import pallas as pl
from jax.experimental.pallas import tpu as pltpu
from jax.experimental.pallas import tpu_sc as plsc

B = 16384
A = 2048
L = 16
NC = 2
NS = 16
NW = NC * NS
RPW = B // NW
CH = 16
NCH = RPW // CH
NCHAIN = 4
SPC = A // L // NCHAIN
CSPAN = SPC * L
UNROLL = 4


def _row_argmax(buf, r):
    lanes = lax.iota(jnp.int32, L)
    accs = [jnp.full((L,), -jnp.inf, jnp.float32) for _ in range(NCHAIN)]
    iaccs = [jnp.zeros((L,), jnp.int32) for _ in range(NCHAIN)]
    jvs = [lanes + c * CSPAN for c in range(NCHAIN)]

    def body(i, carry):
        accs, iaccs, jvs = [list(t) for t in carry]
        base = i * (UNROLL * L)
        for u in range(UNROLL):
            for c in range(NCHAIN):
                val = buf[r, pl.ds(base + u * L + c * CSPAN, L)]
                cmp = val > accs[c]
                accs[c] = jnp.maximum(accs[c], val)
                iaccs[c] = jnp.where(cmp, jvs[c], iaccs[c])
                jvs[c] = jvs[c] + L
        return (tuple(accs), tuple(iaccs), tuple(jvs))

    accs, iaccs, _ = lax.fori_loop(
        0, SPC // UNROLL, body, (tuple(accs), tuple(iaccs), tuple(jvs))
    )
    acc, iacc = accs[0], iaccs[0]
    for c in range(1, NCHAIN):
        take = accs[c] > acc
        acc = jnp.where(take, accs[c], acc)
        iacc = jnp.where(take, iaccs[c], iacc)
    m = jnp.max(acc)
    cand = jnp.where(acc == m, iacc, jnp.full((L,), A, jnp.int32))
    return jnp.min(cand)


def _sc_argmax(x_hbm, out_hbm, buf0, buf1, outv, sem0, sem1):
    c = lax.axis_index("c")
    s = lax.axis_index("s")
    wid = s * NC + c
    base = wid * RPW
    lanes = lax.iota(jnp.int32, L)

    def chunk_src(ci):
        return x_hbm.at[pl.ds(base + ci * CH, CH), 0]

    def compute(buf, ci):
        res = jnp.zeros((L,), jnp.int32)
        for r in range(CH):
            idx = _row_argmax(buf, r)
            res = jnp.where(lanes == r, idx, res)
        outv[pl.ds(ci * CH, L)] = res

    pltpu.make_async_copy(chunk_src(0), buf0, sem0).start()

    def do_pair(p, _):
        ci0 = 2 * p
        ci1 = ci0 + 1
        pltpu.make_async_copy(chunk_src(ci1), buf1, sem1).start()
        pltpu.make_async_copy(chunk_src(ci0), buf0, sem0).wait()
        compute(buf0, ci0)

        @pl.when(p < NCH // 2 - 1)
        def _():
            pltpu.make_async_copy(chunk_src(ci0 + 2), buf0, sem0).start()

        pltpu.make_async_copy(chunk_src(ci1), buf1, sem1).wait()
        compute(buf1, ci1)
        return 0

    lax.fori_loop(0, NCH // 2, do_pair, 0)
    pltpu.sync_copy(outv, out_hbm.at[pl.ds(base, RPW)])


def kernel(pred_actions, target_sizes):
    mesh = plsc.VectorSubcoreMesh(core_axis_name="c", subcore_axis_name="s")
    run = functools.partial(
        pl.kernel,
        mesh=mesh,
        out_type=jax.ShapeDtypeStruct((B,), jnp.int32),
        scratch_types=[
            pltpu.VMEM((CH, A), jnp.float32),
            pltpu.VMEM((CH, A), jnp.float32),
            pltpu.VMEM((RPW,), jnp.int32),
            pltpu.SemaphoreType.DMA,
            pltpu.SemaphoreType.DMA,
        ],
        compiler_params=pltpu.CompilerParams(needs_layout_passes=False),
    )(_sc_argmax)
    return run(pred_actions)


# speedup vs baseline: 1.0559x; 1.0559x over previous
import functools

import jax
import jax.numpy as jnp
from jax import lax
from jax.experimental import pallas as pl
from jax.experimental.pallas import tpu as pltpu
from jax.experimental.pallas import tpu_sc as plsc

B = 16384
A = 2048
L = 16
NC = 2
NS = 16
NW = NC * NS
RPW = B // NW
CH = 16
NCH = RPW // CH
NCHAIN = 4
SPC = A // L // NCHAIN
CSPAN = SPC * L
UNROLL = 2


def _row_argmax(buf, r):
    lanes = lax.iota(jnp.int32, L)
    accs = [jnp.full((L,), -jnp.inf, jnp.float32) for _ in range(NCHAIN)]
    iaccs = [jnp.zeros((L,), jnp.int32) for _ in range(NCHAIN)]
    jvs = [lanes + c * CSPAN for c in range(NCHAIN)]

    def body(i, carry):
        accs, iaccs, jvs = [list(t) for t in carry]
        base = i * (UNROLL * L)
        for u in range(UNROLL):
            for c in range(NCHAIN):
                val = buf[r, pl.ds(base + u * L + c * CSPAN, L)]
                cmp = val > accs[c]
                accs[c] = jnp.maximum(accs[c], val)
                iaccs[c] = jnp.where(cmp, jvs[c], iaccs[c])
                jvs[c] = jvs[c] + L
        return (tuple(accs), tuple(iaccs), tuple(jvs))

    accs, iaccs, _ = lax.fori_loop(
        0, SPC // UNROLL, body, (tuple(accs), tuple(iaccs), tuple(jvs))
    )
    acc, iacc = accs[0], iaccs[0]
    for c in range(1, NCHAIN):
        take = accs[c] > acc
        acc = jnp.where(take, accs[c], acc)
        iacc = jnp.where(take, iaccs[c], iacc)
    m = jnp.max(acc)
    cand = jnp.where(acc == m, iacc, jnp.full((L,), A, jnp.int32))
    return jnp.min(cand)


def _sc_argmax(x_hbm, out_hbm, buf0, buf1, outv, sem0, sem1):
    c = lax.axis_index("c")
    s = lax.axis_index("s")
    wid = s * NC + c
    base = wid * RPW
    lanes = lax.iota(jnp.int32, L)

    def chunk_src(ci):
        return x_hbm.at[pl.ds(base + ci * CH, CH), 0]

    def compute(buf, ci):
        res = jnp.zeros((L,), jnp.int32)
        for r in range(CH):
            idx = _row_argmax(buf, r)
            res = jnp.where(lanes == r, idx, res)
        outv[pl.ds(ci * CH, L)] = res

    pltpu.make_async_copy(chunk_src(0), buf0, sem0).start()

    def do_pair(p, _):
        ci0 = 2 * p
        ci1 = ci0 + 1
        pltpu.make_async_copy(chunk_src(ci1), buf1, sem1).start()
        pltpu.make_async_copy(chunk_src(ci0), buf0, sem0).wait()
        compute(buf0, ci0)

        @pl.when(p < NCH // 2 - 1)
        def _():
            pltpu.make_async_copy(chunk_src(ci0 + 2), buf0, sem0).start()

        pltpu.make_async_copy(chunk_src(ci1), buf1, sem1).wait()
        compute(buf1, ci1)
        return 0

    lax.fori_loop(0, NCH // 2, do_pair, 0)
    pltpu.sync_copy(outv, out_hbm.at[pl.ds(base, RPW)])


def kernel(pred_actions, target_sizes):
    mesh = plsc.VectorSubcoreMesh(core_axis_name="c", subcore_axis_name="s")
    run = functools.partial(
        pl.kernel,
        mesh=mesh,
        out_type=jax.ShapeDtypeStruct((B,), jnp.int32),
        scratch_types=[
            pltpu.VMEM((CH, A), jnp.float32),
            pltpu.VMEM((CH, A), jnp.float32),
            pltpu.VMEM((RPW,), jnp.int32),
            pltpu.SemaphoreType.DMA,
            pltpu.SemaphoreType.DMA,
        ],
        compiler_params=pltpu.CompilerParams(needs_layout_passes=False),
    )(_sc_argmax)
    return run(pred_actions)


# speedup vs baseline: 1.4724x; 1.3944x over previous
import functools

import jax
import jax.numpy as jnp
from jax import lax
from jax.experimental import pallas as pl
from jax.experimental.pallas import tpu as pltpu
from jax.experimental.pallas import tpu_sc as plsc

B = 16384
A = 2048
L = 16
NC = 2
NS = 16
NW = NC * NS
B_SC = 8192
B_TC = B - B_SC
RPW = B_SC // NW
CH = 16
NCH = RPW // CH
NCHAIN = 4
SPC = A // L // NCHAIN
CSPAN = SPC * L
UNROLL = 2
BLK = 512


def _row_argmax(buf, r):
    lanes = lax.iota(jnp.int32, L)
    accs = [jnp.full((L,), -jnp.inf, jnp.float32) for _ in range(NCHAIN)]
    iaccs = [jnp.zeros((L,), jnp.int32) for _ in range(NCHAIN)]
    jvs = [lanes + c * CSPAN for c in range(NCHAIN)]

    def body(i, carry):
        accs, iaccs, jvs = [list(t) for t in carry]
        base = i * (UNROLL * L)
        for u in range(UNROLL):
            for c in range(NCHAIN):
                val = buf[r, pl.ds(base + u * L + c * CSPAN, L)]
                cmp = val > accs[c]
                accs[c] = jnp.maximum(accs[c], val)
                iaccs[c] = jnp.where(cmp, jvs[c], iaccs[c])
                jvs[c] = jvs[c] + L
        return (tuple(accs), tuple(iaccs), tuple(jvs))

    accs, iaccs, _ = lax.fori_loop(
        0, SPC // UNROLL, body, (tuple(accs), tuple(iaccs), tuple(jvs))
    )
    acc, iacc = accs[0], iaccs[0]
    for c in range(1, NCHAIN):
        take = accs[c] > acc
        acc = jnp.where(take, accs[c], acc)
        iacc = jnp.where(take, iaccs[c], iacc)
    m = jnp.max(acc)
    cand = jnp.where(acc == m, iacc, jnp.full((L,), A, jnp.int32))
    return jnp.min(cand)


def _sc_argmax(x_hbm, out_hbm, buf0, buf1, outv, sem0, sem1):
    c = lax.axis_index("c")
    s = lax.axis_index("s")
    wid = s * NC + c
    base = wid * RPW
    lanes = lax.iota(jnp.int32, L)

    def chunk_src(ci):
        return x_hbm.at[pl.ds(base + ci * CH, CH), 0]

    def compute(buf, ci):
        res = jnp.zeros((L,), jnp.int32)
        for r in range(CH):
            idx = _row_argmax(buf, r)
            res = jnp.where(lanes == r, idx, res)
        outv[pl.ds(ci * CH, L)] = res

    pltpu.make_async_copy(chunk_src(0), buf0, sem0).start()

    def do_pair(p, _):
        ci0 = 2 * p
        ci1 = ci0 + 1
        pltpu.make_async_copy(chunk_src(ci1), buf1, sem1).start()
        pltpu.make_async_copy(chunk_src(ci0), buf0, sem0).wait()
        compute(buf0, ci0)

        @pl.when(p < NCH // 2 - 1)
        def _():
            pltpu.make_async_copy(chunk_src(ci0 + 2), buf0, sem0).start()

        pltpu.make_async_copy(chunk_src(ci1), buf1, sem1).wait()
        compute(buf1, ci1)
        return 0

    lax.fori_loop(0, NCH // 2, do_pair, 0)
    pltpu.sync_copy(outv, out_hbm.at[pl.ds(base, RPW)])


def _tc_body(x_ref, o_ref):
    x = x_ref[...].reshape(BLK, A)
    m = jnp.max(x, axis=1, keepdims=True)
    ids = lax.broadcasted_iota(jnp.int32, (BLK, A), 1)
    cand = jnp.where(x == m, ids, A)
    o_ref[...] = jnp.min(cand, axis=1)


def kernel(pred_actions, target_sizes):
    mesh = plsc.VectorSubcoreMesh(core_axis_name="c", subcore_axis_name="s")
    sc_run = functools.partial(
        pl.kernel,
        mesh=mesh,
        out_type=jax.ShapeDtypeStruct((B_SC,), jnp.int32),
        scratch_types=[
            pltpu.VMEM((CH, A), jnp.float32),
            pltpu.VMEM((CH, A), jnp.float32),
            pltpu.VMEM((RPW,), jnp.int32),
            pltpu.SemaphoreType.DMA,
            pltpu.SemaphoreType.DMA,
        ],
        compiler_params=pltpu.CompilerParams(needs_layout_passes=False),
    )(_sc_argmax)
    out_sc = sc_run(pred_actions)

    tc_run = pl.pallas_call(
        _tc_body,
        grid=(B_TC // BLK,),
        in_specs=[
            pl.BlockSpec((BLK, 1, A), lambda i: (B_SC // BLK + i, 0, 0)),
        ],
        out_specs=pl.BlockSpec((BLK,), lambda i: (i,)),
        out_shape=jax.ShapeDtypeStruct((B_TC,), jnp.int32),
    )
    out_tc = tc_run(pred_actions)
    return jnp.concatenate([out_sc, out_tc])
